# Initial kernel scaffold; baseline (speedup 1.0000x reference)
import jax
import jax.numpy as jnp
from jax.experimental import pallas as pl


def kernel(non_seq_features, tables, W_user, W_item):
    raise NotImplementedError("write your pallas kernel here")


# speedup vs baseline: 1.9161x; 1.9161x over previous
import functools

import jax
import jax.numpy as jnp
from jax import lax
from jax.experimental import pallas as pl
from jax.experimental.pallas import tpu as pltpu
from jax.experimental.pallas import tpu_sc as plsc

NUM_FEATURES = 26
VOCAB = 100000
EMBED_DIM = 32
BATCH = 16384
HEADS = 8
SPLIT = 104
HEAD_DIM = 64
D_ALL = NUM_FEATURES * EMBED_DIM

NC, NS = 2, 16
NW = NC * NS
ROWS = NUM_FEATURES * BATCH
RPW = ROWS // NW
CH = 1024
NCH = RPW // CH

_mesh = plsc.VectorSubcoreMesh(core_axis_name="c", subcore_axis_name="s")


@functools.partial(
    pl.kernel,
    mesh=_mesh,
    compiler_params=pltpu.CompilerParams(use_tc_tiling_on_sc=False),
    out_type=jax.ShapeDtypeStruct((ROWS, EMBED_DIM), jnp.float32),
    scratch_types=[
        pltpu.VMEM((CH,), jnp.int32),
        pltpu.VMEM((CH,), jnp.int32),
        pltpu.VMEM((CH, EMBED_DIM), jnp.float32),
        pltpu.VMEM((CH, EMBED_DIM), jnp.float32),
        pltpu.SemaphoreType.DMA,
        pltpu.SemaphoreType.DMA,
        pltpu.SemaphoreType.DMA,
    ],
)
def _sc_gather(tab_hbm, idx_hbm, out_hbm, idx0, idx1, buf0, buf1,
               isem, gsem, wsem):
    wid = lax.axis_index("s") * NC + lax.axis_index("c")
    base = wid * RPW
    idx_bufs = [idx0, idx1]
    row_bufs = [buf0, buf1]
    pltpu.sync_copy(idx_hbm.at[pl.ds(base, CH)], idx0)
    writes = [None, None]
    for c in range(NCH):
        b = c % 2
        nb = (c + 1) % 2
        nxt = None
        if c + 1 < NCH:
            nxt = pltpu.async_copy(
                idx_hbm.at[pl.ds(base + (c + 1) * CH, CH)], idx_bufs[nb], isem)
        if writes[b] is not None:
            writes[b].wait()
        pltpu.async_copy(tab_hbm.at[idx_bufs[b]], row_bufs[b], gsem).wait()
        writes[b] = pltpu.async_copy(
            row_bufs[b], out_hbm.at[pl.ds(base + c * CH, CH)], wsem)
        if nxt is not None:
            nxt.wait()
    for w in writes:
        if w is not None:
            w.wait()


def _mm_body(x_ref, w_ref, o_ref):
    x = x_ref[...]
    for j in range(HEADS):
        o_ref[:, j * HEAD_DIM:(j + 1) * HEAD_DIM] = jnp.dot(
            x[:, j * SPLIT:(j + 1) * SPLIT], w_ref[j],
            preferred_element_type=jnp.float32)


BB = 1024

_mm = pl.pallas_call(
    _mm_body,
    grid=(BATCH // BB,),
    in_specs=[
        pl.BlockSpec((BB, D_ALL), lambda i: (i, 0)),
        pl.BlockSpec((HEADS, SPLIT, HEAD_DIM), lambda i: (0, 0, 0)),
    ],
    out_specs=pl.BlockSpec((BB, HEADS * HEAD_DIM), lambda i: (i, 0)),
    out_shape=jax.ShapeDtypeStruct((BATCH, HEADS * HEAD_DIM), jnp.float32),
)


def kernel(non_seq_features, tables, W_user, W_item):
    offs = (jnp.arange(NUM_FEATURES, dtype=jnp.int32) * VOCAB)[:, None]
    gidx = (non_seq_features + offs).T.reshape(ROWS)
    flat = tables.reshape(NUM_FEATURES * VOCAB, EMBED_DIM)
    gathered = _sc_gather(flat, gidx)
    wt = jnp.concatenate([W_user, W_item], axis=0).transpose(0, 2, 1)
    out = _mm(gathered.reshape(BATCH, D_ALL), wt)
    return out.reshape(BATCH, HEADS, HEAD_DIM)


# speedup vs baseline: 1.9179x; 1.0010x over previous
import functools

import jax
import jax.numpy as jnp
from jax import lax
from jax.experimental import pallas as pl
from jax.experimental.pallas import tpu as pltpu
from jax.experimental.pallas import tpu_sc as plsc

NUM_FEATURES = 26
VOCAB = 100000
EMBED_DIM = 32
BATCH = 16384
HEADS = 8
SPLIT = 104
HEAD_DIM = 64
D_ALL = NUM_FEATURES * EMBED_DIM

NC, NS = 2, 16
NW = NC * NS
ROWS = NUM_FEATURES * BATCH
RPW = ROWS // NW
CH = 1024
NCH = RPW // CH

_mesh = plsc.VectorSubcoreMesh(core_axis_name="c", subcore_axis_name="s")


BPW = BATCH // NW


@functools.partial(
    pl.kernel,
    mesh=_mesh,
    compiler_params=pltpu.CompilerParams(
        use_tc_tiling_on_sc=False, needs_layout_passes=False),
    out_type=jax.ShapeDtypeStruct((ROWS, EMBED_DIM), jnp.float32),
    scratch_types=[
        pltpu.VMEM((RPW,), jnp.int32),
        pltpu.VMEM((RPW,), jnp.int32),
        pltpu.VMEM((CH, EMBED_DIM), jnp.float32),
        pltpu.VMEM((CH, EMBED_DIM), jnp.float32),
        pltpu.SemaphoreType.DMA,
        pltpu.SemaphoreType.DMA,
        pltpu.SemaphoreType.DMA,
    ],
)
def _sc_gather(tab_hbm, nsf_hbm, out_hbm, raw_v, gidx_v, buf0, buf1,
               isem, gsem, wsem):
    wid = lax.axis_index("s") * NC + lax.axis_index("c")
    b0 = wid * BPW
    base = wid * RPW
    loads = []
    for f in range(NUM_FEATURES):
        loads.append(pltpu.async_copy(
            nsf_hbm.at[pl.ds(f * BATCH + b0, BPW)],
            raw_v.at[pl.ds(f * BPW, BPW)], isem))
    for ld in loads:
        ld.wait()

    iota26 = lax.iota(jnp.int32, 16) * NUM_FEATURES

    def build(k, carry):
        for f in range(NUM_FEATURES):
            v = raw_v[pl.ds(f * BPW + k * 16, 16)] + (f * VOCAB)
            pos = iota26 + (k * (16 * NUM_FEATURES) + f)
            plsc.store_scatter(gidx_v, [pos], v)
        return carry
    lax.fori_loop(0, BPW // 16, build, 0)

    writes = [None, None]
    bufs = [buf0, buf1]
    for c in range(NCH):
        bb = c % 2
        if writes[bb] is not None:
            writes[bb].wait()
        pltpu.async_copy(
            tab_hbm.at[gidx_v.at[pl.ds(c * CH, CH)]], bufs[bb], gsem).wait()
        writes[bb] = pltpu.async_copy(
            bufs[bb], out_hbm.at[pl.ds(base + c * CH, CH)], wsem)
    for w in writes:
        if w is not None:
            w.wait()


def _mm_body(x_ref, w_ref, o_ref):
    x = x_ref[...]
    for j in range(HEADS):
        o_ref[:, j * HEAD_DIM:(j + 1) * HEAD_DIM] = jnp.dot(
            x[:, j * SPLIT:(j + 1) * SPLIT], w_ref[j],
            preferred_element_type=jnp.float32)


BB = 1024

_mm = pl.pallas_call(
    _mm_body,
    grid=(BATCH // BB,),
    in_specs=[
        pl.BlockSpec((BB, D_ALL), lambda i: (i, 0)),
        pl.BlockSpec((HEADS, SPLIT, HEAD_DIM), lambda i: (0, 0, 0)),
    ],
    out_specs=pl.BlockSpec((BB, HEADS * HEAD_DIM), lambda i: (i, 0)),
    out_shape=jax.ShapeDtypeStruct((BATCH, HEADS * HEAD_DIM), jnp.float32),
)


def kernel(non_seq_features, tables, W_user, W_item):
    flat = tables.reshape(NUM_FEATURES * VOCAB, EMBED_DIM)
    nsf = non_seq_features.reshape(ROWS)
    gathered = _sc_gather(flat, nsf)
    wt = jnp.concatenate([W_user, W_item], axis=0).transpose(0, 2, 1)
    out = _mm(gathered.reshape(BATCH, D_ALL), wt)
    return out.reshape(BATCH, HEADS, HEAD_DIM)


# speedup vs baseline: 1.9332x; 1.0080x over previous
import functools

import jax
import jax.numpy as jnp
from jax import lax
from jax.experimental import pallas as pl
from jax.experimental.pallas import tpu as pltpu
from jax.experimental.pallas import tpu_sc as plsc

NUM_FEATURES = 26
VOCAB = 100000
EMBED_DIM = 32
BATCH = 16384
HEADS = 8
SPLIT = 104
HEAD_DIM = 64
D_ALL = NUM_FEATURES * EMBED_DIM

NC, NS = 2, 16
NW = NC * NS
ROWS = NUM_FEATURES * BATCH
RPW = ROWS // NW
CH = 1024
NCH = RPW // CH

_mesh = plsc.VectorSubcoreMesh(core_axis_name="c", subcore_axis_name="s")


BPW = BATCH // NW


@functools.partial(
    pl.kernel,
    mesh=_mesh,
    compiler_params=pltpu.CompilerParams(
        use_tc_tiling_on_sc=False, needs_layout_passes=False),
    out_type=jax.ShapeDtypeStruct((ROWS, EMBED_DIM), jnp.float32),
    scratch_types=[
        pltpu.VMEM((RPW,), jnp.int32),
        pltpu.VMEM((RPW,), jnp.int32),
        pltpu.VMEM((CH, EMBED_DIM), jnp.float32),
        pltpu.VMEM((CH, EMBED_DIM), jnp.float32),
        pltpu.SemaphoreType.DMA,
        pltpu.SemaphoreType.DMA,
        pltpu.SemaphoreType.DMA,
    ],
)
def _sc_gather(tab_hbm, nsf_hbm, out_hbm, raw_v, gidx_v, buf0, buf1,
               isem, gsem, wsem):
    wid = lax.axis_index("s") * NC + lax.axis_index("c")
    b0 = wid * BPW
    base = wid * RPW
    loads = []
    for f in range(NUM_FEATURES):
        loads.append(pltpu.async_copy(
            nsf_hbm.at[pl.ds(f * BATCH + b0, BPW)],
            raw_v.at[pl.ds(f * BPW, BPW)], isem))
    for ld in loads:
        ld.wait()

    iota26 = lax.iota(jnp.int32, 16) * NUM_FEATURES

    def build(k, carry):
        for f in range(NUM_FEATURES):
            v = raw_v[pl.ds(f * BPW + k * 16, 16)] + (f * VOCAB)
            pos = iota26 + (k * (16 * NUM_FEATURES) + f)
            plsc.store_scatter(gidx_v, [pos], v)
        return carry
    lax.fori_loop(0, BPW // 16, build, 0)

    writes = [None, None]
    bufs = [buf0, buf1]
    for c in range(NCH):
        bb = c % 2
        if writes[bb] is not None:
            writes[bb].wait()
        pltpu.async_copy(
            tab_hbm.at[gidx_v.at[pl.ds(c * CH, CH)]], bufs[bb], gsem).wait()
        writes[bb] = pltpu.async_copy(
            bufs[bb], out_hbm.at[pl.ds(base + c * CH, CH)], wsem)
    for w in writes:
        if w is not None:
            w.wait()


def _mm_body(x_ref, w_ref, o_ref):
    o_ref[...] = jnp.dot(x_ref[...], w_ref[...],
                         preferred_element_type=jnp.float32)


BB = 1024

_mm = pl.pallas_call(
    _mm_body,
    grid=(BATCH // BB,),
    in_specs=[
        pl.BlockSpec((BB, D_ALL), lambda i: (i, 0)),
        pl.BlockSpec((D_ALL, HEADS * HEAD_DIM), lambda i: (0, 0)),
    ],
    out_specs=pl.BlockSpec((BB, HEADS * HEAD_DIM), lambda i: (i, 0)),
    out_shape=jax.ShapeDtypeStruct((BATCH, HEADS * HEAD_DIM), jnp.float32),
)


def kernel(non_seq_features, tables, W_user, W_item):
    flat = tables.reshape(NUM_FEATURES * VOCAB, EMBED_DIM)
    nsf = non_seq_features.reshape(ROWS)
    gathered = _sc_gather(flat, nsf)
    wt = jnp.concatenate([W_user, W_item], axis=0).transpose(0, 2, 1)
    wbd = jax.scipy.linalg.block_diag(*[wt[j] for j in range(HEADS)])
    out = _mm(gathered.reshape(BATCH, D_ALL), wbd)
    return out.reshape(BATCH, HEADS, HEAD_DIM)


# speedup vs baseline: 1.9792x; 1.0238x over previous
import functools

import jax
import jax.numpy as jnp
from jax import lax
from jax.experimental import pallas as pl
from jax.experimental.pallas import tpu as pltpu
from jax.experimental.pallas import tpu_sc as plsc

NUM_FEATURES = 26
VOCAB = 100000
EMBED_DIM = 32
BATCH = 16384
HEADS = 8
SPLIT = 104
HEAD_DIM = 64
D_ALL = NUM_FEATURES * EMBED_DIM

NC, NS = 2, 16
NW = NC * NS
ROWS = NUM_FEATURES * BATCH
RPW = ROWS // NW
CH = 1024
NCH = RPW // CH

_mesh = plsc.VectorSubcoreMesh(core_axis_name="c", subcore_axis_name="s")


BPW = BATCH // NW


@functools.partial(
    pl.kernel,
    mesh=_mesh,
    compiler_params=pltpu.CompilerParams(
        use_tc_tiling_on_sc=False, needs_layout_passes=False),
    out_type=jax.ShapeDtypeStruct((ROWS, EMBED_DIM), jnp.float32),
    scratch_types=[
        pltpu.VMEM((RPW,), jnp.int32),
        pltpu.VMEM((RPW,), jnp.int32),
        pltpu.VMEM((CH, EMBED_DIM), jnp.float32),
        pltpu.VMEM((CH, EMBED_DIM), jnp.float32),
        pltpu.SemaphoreType.DMA,
        pltpu.SemaphoreType.DMA,
        pltpu.SemaphoreType.DMA,
    ],
)
def _sc_gather(tab_hbm, nsf_hbm, out_hbm, raw_v, gidx_v, buf0, buf1,
               isem, gsem, wsem):
    wid = lax.axis_index("s") * NC + lax.axis_index("c")
    b0 = wid * BPW
    base = wid * RPW
    loads = []
    for f in range(NUM_FEATURES):
        loads.append(pltpu.async_copy(
            nsf_hbm.at[pl.ds(f * BATCH + b0, BPW)],
            raw_v.at[pl.ds(f * BPW, BPW)], isem))
    for ld in loads:
        ld.wait()

    iota26 = lax.iota(jnp.int32, 16) * NUM_FEATURES

    def build(k, carry):
        for f in range(NUM_FEATURES):
            v = raw_v[pl.ds(f * BPW + k * 16, 16)] + (f * VOCAB)
            pos = iota26 + (k * (16 * NUM_FEATURES) + f)
            plsc.store_scatter(gidx_v, [pos], v)
        return carry
    lax.fori_loop(0, BPW // 16, build, 0)

    writes = [None, None]
    bufs = [buf0, buf1]
    for c in range(NCH):
        bb = c % 2
        if writes[bb] is not None:
            writes[bb].wait()
        pltpu.async_copy(
            tab_hbm.at[gidx_v.at[pl.ds(c * CH, CH)]], bufs[bb], gsem).wait()
        writes[bb] = pltpu.async_copy(
            bufs[bb], out_hbm.at[pl.ds(base + c * CH, CH)], wsem)
    for w in writes:
        if w is not None:
            w.wait()


def _mm_body(x_ref, w_ref, o_ref):
    o_ref[...] = jax.lax.dot_general(
        w_ref[...], x_ref[...],
        dimension_numbers=(((1,), (1,)), ((), ())),
        preferred_element_type=jnp.float32)


BB = 1024

_mm = pl.pallas_call(
    _mm_body,
    grid=(BATCH // BB,),
    in_specs=[
        pl.BlockSpec((BB, D_ALL), lambda i: (i, 0)),
        pl.BlockSpec((HEADS * HEAD_DIM, D_ALL), lambda i: (0, 0)),
    ],
    out_specs=pl.BlockSpec((HEADS * HEAD_DIM, BB), lambda i: (0, i)),
    out_shape=jax.ShapeDtypeStruct((HEADS * HEAD_DIM, BATCH), jnp.float32),
)


def kernel(non_seq_features, tables, W_user, W_item):
    flat = tables.reshape(NUM_FEATURES * VOCAB, EMBED_DIM)
    nsf = non_seq_features.reshape(ROWS)
    gathered = _sc_gather(flat, nsf)
    w_all = jnp.concatenate([W_user, W_item], axis=0)
    wbd_t = jax.scipy.linalg.block_diag(*[w_all[j] for j in range(HEADS)])
    out_t = _mm(gathered.reshape(BATCH, D_ALL), wbd_t)
    return out_t.reshape(HEADS, HEAD_DIM, BATCH).transpose(2, 0, 1)
